# baseline (speedup 1.0000x reference)
import functools

import jax
import jax.numpy as jnp
from jax import lax
from jax.experimental import pallas as pl
from jax.experimental.pallas import tpu as pltpu
from jax.experimental.pallas import tpu_sc as plsc

TC_ROWS = 512


def _sc_gather(ids_sc, table):
    (S,) = ids_sc.shape
    V, D = table.shape
    info = plsc.get_sparse_core_info()
    nc, ns = info.num_cores, info.num_subcores
    nw = nc * ns
    b_per_w = S // nw
    rows_per_chunk = 8
    n_chunks = b_per_w // rows_per_chunk

    mesh = plsc.VectorSubcoreMesh(core_axis_name="c", subcore_axis_name="s")

    @functools.partial(
        pl.kernel,
        mesh=mesh,
        out_type=jax.ShapeDtypeStruct((S, D), jnp.float32),
        scratch_types=[
            pltpu.VMEM((b_per_w,), jnp.int32),
            pltpu.VMEM((b_per_w, D), jnp.float32),
            pltpu.SemaphoreType.DMA((b_per_w // 8,)),
            pltpu.SemaphoreType.DMA((b_per_w // 8,)),
        ],
    )
    def emb(idx_hbm, table_hbm, out_hbm, idx_v, rows_v, gsem, wsem):
        wid = lax.axis_index("s") * nc + lax.axis_index("c")
        base = wid * b_per_w
        pltpu.sync_copy(idx_hbm.at[pl.ds(base, b_per_w)], idx_v)
        gathers = []
        for c in range(n_chunks):
            r0 = c * rows_per_chunk
            cp = pltpu.make_async_copy(
                table_hbm.at[idx_v.at[pl.ds(r0, rows_per_chunk)]],
                rows_v.at[pl.ds(r0, rows_per_chunk)],
                gsem.at[c],
            )
            cp.start()
            gathers.append(cp)
        writes = []
        for c in range(n_chunks):
            r0 = c * rows_per_chunk
            gathers[c].wait()
            cp = pltpu.make_async_copy(
                rows_v.at[pl.ds(r0, rows_per_chunk)],
                out_hbm.at[pl.ds(base + r0, rows_per_chunk)],
                wsem.at[c],
            )
            cp.start()
            writes.append(cp)
        for c in range(n_chunks):
            writes[c].wait()

    return emb(ids_sc, table)


def _tc_gather(ids_tc, table):
    (S,) = ids_tc.shape
    V, D = table.shape

    def body(ids_ref, table_ref, out_ref, sem):
        def start(i, _):
            idx = ids_ref[i]
            pltpu.make_async_copy(
                table_ref.at[pl.ds(idx, 1)], out_ref.at[pl.ds(i, 1)], sem
            ).start()
            return 0
        lax.fori_loop(0, S, start, 0)

        def drain(i, _):
            pltpu.make_async_copy(
                table_ref.at[pl.ds(0, 1)], out_ref.at[pl.ds(i, 1)], sem
            ).wait()
            return 0
        lax.fori_loop(0, S, drain, 0)

    return pl.pallas_call(
        body,
        in_specs=[
            pl.BlockSpec(memory_space=pltpu.MemorySpace.SMEM),
            pl.BlockSpec(memory_space=pltpu.MemorySpace.HBM),
        ],
        out_specs=pl.BlockSpec(memory_space=pltpu.MemorySpace.HBM),
        out_shape=jax.ShapeDtypeStruct((S, D), jnp.float32),
        scratch_shapes=[pltpu.SemaphoreType.DMA],
    )(ids_tc, table)


def kernel(input_ids, embed_table):
    ids = input_ids.astype(jnp.int32)
    head = _tc_gather(ids[:TC_ROWS], embed_table)
    tail = _sc_gather(ids[TC_ROWS:], embed_table)
    return jnp.concatenate([head, tail], axis=0)


# speedup vs baseline: 3.4171x; 3.4171x over previous
import functools

import jax
import jax.numpy as jnp
from jax import lax
from jax.experimental import pallas as pl
from jax.experimental.pallas import tpu as pltpu
from jax.experimental.pallas import tpu_sc as plsc


def kernel(input_ids, embed_table):
    (B,) = input_ids.shape
    V, D = embed_table.shape

    info = plsc.get_sparse_core_info()
    nc, ns = info.num_cores, info.num_subcores
    nw = nc * ns
    b_per_w = B // nw

    mesh = plsc.VectorSubcoreMesh(core_axis_name="c", subcore_axis_name="s")

    @functools.partial(
        pl.kernel,
        mesh=mesh,
        out_type=jax.ShapeDtypeStruct((B, D), jnp.float32),
        scratch_types=[
            pltpu.VMEM((b_per_w,), jnp.int32),
            pltpu.VMEM((b_per_w, D), jnp.float32),
            pltpu.SemaphoreType.DMA,
        ],
    )
    def emb(idx_hbm, table_hbm, out_hbm, idx_v, rows_v, sem):
        wid = lax.axis_index("s") * nc + lax.axis_index("c")
        base = wid * b_per_w
        pltpu.sync_copy(idx_hbm.at[pl.ds(base, b_per_w)], idx_v)
        pltpu.async_copy(table_hbm.at[idx_v], rows_v, sem).wait()
        pltpu.sync_copy(rows_v, out_hbm.at[pl.ds(base, b_per_w)])

    return emb(input_ids.astype(jnp.int32), embed_table)
